# baseline (speedup 1.0000x reference)
import jax
import jax.numpy as jnp
from jax.experimental import pallas as pl


def kernel(x, i, t, pad_lengths, decoder_embedding_weight):
    s, b, c = x.shape
    dt, _ = decoder_embedding_weight.shape
    dims = i.shape[2]
    BLK = 128
    n_dt = dt // BLK
    n_total = (dt + s) // BLK

    t2 = t.reshape(1, b)
    pad2 = pad_lengths.reshape(1, b)

    def body(w_ref, x_ref, i_ref, t_ref, pad_ref, xo_ref, io_ref, po_ref):
        p = pl.program_id(0)

        @pl.when(p < n_dt)
        def _():
            xo_ref[...] = jnp.broadcast_to(w_ref[...][:, None, :], (BLK, b, c))
            d_idx = jax.lax.broadcasted_iota(jnp.int32, (BLK, b, dims), 2)
            tv = t_ref[...].reshape(1, b, 1)
            io_ref[...] = jnp.where(d_idx == 0, 1, jnp.where(d_idx == 1, tv, -1))

        @pl.when(p >= n_dt)
        def _():
            xo_ref[...] = x_ref[...]
            io_ref[...] = i_ref[...]

        po_ref[...] = pad_ref[...] + dt

    grid = (n_total,)
    in_specs = [
        pl.BlockSpec((BLK, c), lambda p: (jnp.minimum(p, n_dt - 1), 0)),
        pl.BlockSpec((BLK, b, c), lambda p: (jnp.maximum(p - n_dt, 0), 0, 0)),
        pl.BlockSpec((BLK, b, dims), lambda p: (jnp.maximum(p - n_dt, 0), 0, 0)),
        pl.BlockSpec((1, b), lambda p: (0, 0)),
        pl.BlockSpec((1, b), lambda p: (0, 0)),
    ]
    out_specs = [
        pl.BlockSpec((BLK, b, c), lambda p: (p, 0, 0)),
        pl.BlockSpec((BLK, b, dims), lambda p: (p, 0, 0)),
        pl.BlockSpec((1, b), lambda p: (0, 0)),
    ]
    out_shape = [
        jax.ShapeDtypeStruct((dt + s, b, c), x.dtype),
        jax.ShapeDtypeStruct((dt + s, b, dims), i.dtype),
        jax.ShapeDtypeStruct((1, b), pad_lengths.dtype),
    ]
    xo, io, po = pl.pallas_call(
        body, grid=grid, in_specs=in_specs, out_specs=out_specs,
        out_shape=out_shape,
    )(decoder_embedding_weight, x, i, t2, pad2)
    return xo, io, po.reshape(b)


# speedup vs baseline: 2.9451x; 2.9451x over previous
import jax
import jax.numpy as jnp
from jax.experimental import pallas as pl


def kernel(x, i, t, pad_lengths, decoder_embedding_weight):
    s, b, c = x.shape
    dt, _ = decoder_embedding_weight.shape
    dims = i.shape[2]
    bd = b * dims
    BLK = 128
    n_dt = dt // BLK
    n_total = (dt + s) // BLK

    t2 = t.reshape(1, b)
    pad2 = pad_lengths.reshape(1, b)
    i2 = i.reshape(s, bd)

    def body(w_ref, x_ref, i_ref, t_ref, pad_ref, xo_ref, io_ref, po_ref):
        p = pl.program_id(0)

        @pl.when(p < n_dt)
        def _():
            xo_ref[...] = jnp.broadcast_to(w_ref[...][:, None, :], (BLK, b, c))
            lane = jax.lax.broadcasted_iota(jnp.int32, (1, bd), 1)
            tv = jnp.repeat(t_ref[...], dims, axis=1)
            row = jnp.where(lane % dims == 0, 1,
                            jnp.where(lane % dims == 1, tv, -1))
            io_ref[...] = jnp.broadcast_to(row, (BLK, bd))

        @pl.when(p >= n_dt)
        def _():
            xo_ref[...] = x_ref[...]
            io_ref[...] = i_ref[...]

        po_ref[...] = pad_ref[...] + dt

    grid = (n_total,)
    in_specs = [
        pl.BlockSpec((BLK, c), lambda p: (jnp.minimum(p, n_dt - 1), 0)),
        pl.BlockSpec((BLK, b, c), lambda p: (jnp.maximum(p - n_dt, 0), 0, 0)),
        pl.BlockSpec((BLK, bd), lambda p: (jnp.maximum(p - n_dt, 0), 0)),
        pl.BlockSpec((1, b), lambda p: (0, 0)),
        pl.BlockSpec((1, b), lambda p: (0, 0)),
    ]
    out_specs = [
        pl.BlockSpec((BLK, b, c), lambda p: (p, 0, 0)),
        pl.BlockSpec((BLK, bd), lambda p: (p, 0)),
        pl.BlockSpec((1, b), lambda p: (0, 0)),
    ]
    out_shape = [
        jax.ShapeDtypeStruct((dt + s, b, c), x.dtype),
        jax.ShapeDtypeStruct((dt + s, bd), i.dtype),
        jax.ShapeDtypeStruct((1, b), pad_lengths.dtype),
    ]
    xo, io, po = pl.pallas_call(
        body, grid=grid, in_specs=in_specs, out_specs=out_specs,
        out_shape=out_shape,
    )(decoder_embedding_weight, x, i2, t2, pad2)
    return xo, io.reshape(dt + s, b, dims), po.reshape(b)


# speedup vs baseline: 2.9626x; 1.0059x over previous
import jax
import jax.numpy as jnp
from jax.experimental import pallas as pl


def kernel(x, i, t, pad_lengths, decoder_embedding_weight):
    s, b, c = x.shape
    dt, _ = decoder_embedding_weight.shape
    dims = i.shape[2]
    bd = b * dims
    BLK = 256
    n_dt = dt // BLK
    n_total = (dt + s) // BLK

    t2 = t.reshape(1, b)
    pad2 = pad_lengths.reshape(1, b)
    i2 = i.reshape(s, bd)

    def body(w_ref, x_ref, i_ref, t_ref, pad_ref, xo_ref, io_ref, po_ref):
        p = pl.program_id(0)

        @pl.when(p < n_dt)
        def _():
            xo_ref[...] = jnp.broadcast_to(w_ref[...][:, None, :], (BLK, b, c))
            lane = jax.lax.broadcasted_iota(jnp.int32, (1, bd), 1)
            tv = jnp.repeat(t_ref[...], dims, axis=1)
            row = jnp.where(lane % dims == 0, 1,
                            jnp.where(lane % dims == 1, tv, -1))
            io_ref[...] = jnp.broadcast_to(row, (BLK, bd))

        @pl.when(p >= n_dt)
        def _():
            xo_ref[...] = x_ref[...]
            io_ref[...] = i_ref[...]

        po_ref[...] = pad_ref[...] + dt

    grid = (n_total,)
    in_specs = [
        pl.BlockSpec((BLK, c), lambda p: (jnp.minimum(p, n_dt - 1), 0)),
        pl.BlockSpec((BLK, b, c), lambda p: (jnp.maximum(p - n_dt, 0), 0, 0)),
        pl.BlockSpec((BLK, bd), lambda p: (jnp.maximum(p - n_dt, 0), 0)),
        pl.BlockSpec((1, b), lambda p: (0, 0)),
        pl.BlockSpec((1, b), lambda p: (0, 0)),
    ]
    out_specs = [
        pl.BlockSpec((BLK, b, c), lambda p: (p, 0, 0)),
        pl.BlockSpec((BLK, bd), lambda p: (p, 0)),
        pl.BlockSpec((1, b), lambda p: (0, 0)),
    ]
    out_shape = [
        jax.ShapeDtypeStruct((dt + s, b, c), x.dtype),
        jax.ShapeDtypeStruct((dt + s, bd), i.dtype),
        jax.ShapeDtypeStruct((1, b), pad_lengths.dtype),
    ]
    xo, io, po = pl.pallas_call(
        body, grid=grid, in_specs=in_specs, out_specs=out_specs,
        out_shape=out_shape,
    )(decoder_embedding_weight, x, i2, t2, pad2)
    return xo, io.reshape(dt + s, b, dims), po.reshape(b)
